# Initial kernel scaffold; baseline (speedup 1.0000x reference)
import jax
import jax.numpy as jnp
from jax.experimental import pallas as pl


def kernel(x, edge_index, W_enc, b_enc, Wl, bl, ln_s, ln_b, lnf_s, lnf_b, W1, b1, W2, b2):
    raise NotImplementedError("write your pallas kernel here")


# speedup vs baseline: 3.5728x; 3.5728x over previous
import functools

import jax
import jax.numpy as jnp
from jax import lax
from jax.experimental import pallas as pl
from jax.experimental.pallas import tpu as pltpu
from jax.experimental.pallas import tpu_sc as plsc

N = 10000
E = 320000
H = 128
NC = 2
NS = 16
NW = NC * NS
PER_W = E // NW
CH = 80
N_CH = PER_W // CH
ST = 624
REM = N - NS * ST

_sc_mesh = plsc.VectorSubcoreMesh(core_axis_name="c", subcore_axis_name="s",
                                  num_cores=NC, num_subcores=NS)


def _sc_segsum(t, src, dst, zeros):
    @functools.partial(
        pl.kernel,
        out_type=jax.ShapeDtypeStruct((NC * N, H), jnp.float32),
        mesh=_sc_mesh,
        scratch_types=[
            pltpu.VMEM((CH,), jnp.int32),
            pltpu.VMEM((CH,), jnp.int32),
            pltpu.VMEM((CH, H), jnp.float32),
            pltpu.VMEM_SHARED((N, H), jnp.float32),
            pltpu.SemaphoreType.DMA,
        ],
    )
    def k(t_hbm, src_hbm, dst_hbm, z_hbm, out_hbm, sidx, didx, rows, acc, sem):
        cid = lax.axis_index("c")
        sid = lax.axis_index("s")
        pltpu.sync_copy(z_hbm.at[pl.ds(sid * ST, ST)],
                        acc.at[pl.ds(sid * ST, ST)])

        @pl.when(sid == NS - 1)
        def _():
            pltpu.sync_copy(z_hbm.at[pl.ds(NS * ST, REM)],
                            acc.at[pl.ds(NS * ST, REM)])

        plsc.subcore_barrier()
        base = (cid * NS + sid) * PER_W

        @pl.loop(0, N_CH)
        def _(c):
            off = base + c * CH
            pltpu.sync_copy(src_hbm.at[pl.ds(off, CH)], sidx)
            pltpu.sync_copy(dst_hbm.at[pl.ds(off, CH)], didx)
            pltpu.async_copy(t_hbm.at[sidx], rows, sem).wait()
            pltpu.sync_copy(rows, acc.at[didx], add=True)

        plsc.subcore_barrier()
        pltpu.sync_copy(acc.at[pl.ds(sid * ST, ST)],
                        out_hbm.at[pl.ds(cid * N + sid * ST, ST)])

        @pl.when(sid == NS - 1)
        def _():
            pltpu.sync_copy(acc.at[pl.ds(NS * ST, REM)],
                            out_hbm.at[pl.ds(cid * N + NS * ST, REM)])

    return k(t, src, dst, zeros)


def _sc_gather2(a, b, src, dst):
    @functools.partial(
        pl.kernel,
        out_type=[jax.ShapeDtypeStruct((E, H), jnp.float32),
                  jax.ShapeDtypeStruct((E, H), jnp.float32)],
        mesh=_sc_mesh,
        scratch_types=[
            pltpu.VMEM((CH,), jnp.int32),
            pltpu.VMEM((CH,), jnp.int32),
            pltpu.VMEM((CH, H), jnp.float32),
            pltpu.VMEM((CH, H), jnp.float32),
            pltpu.SemaphoreType.DMA,
            pltpu.SemaphoreType.DMA,
        ],
    )
    def k(a_hbm, b_hbm, src_hbm, dst_hbm, r1_hbm, r2_hbm,
          sidx, didx, buf1, buf2, sem1, sem2):
        cid = lax.axis_index("c")
        sid = lax.axis_index("s")
        base = (cid * NS + sid) * PER_W

        @pl.loop(0, N_CH)
        def _(c):
            off = base + c * CH
            pltpu.sync_copy(src_hbm.at[pl.ds(off, CH)], sidx)
            pltpu.sync_copy(dst_hbm.at[pl.ds(off, CH)], didx)
            cp1 = pltpu.async_copy(a_hbm.at[sidx], buf1, sem1)
            cp2 = pltpu.async_copy(b_hbm.at[didx], buf2, sem2)
            cp1.wait()
            cp2.wait()
            pltpu.sync_copy(buf1, r1_hbm.at[pl.ds(off, CH)])
            pltpu.sync_copy(buf2, r2_hbm.at[pl.ds(off, CH)])

    return k(a, b, src, dst)


_RB = 2000
_EB = 2000


def _ln_relu(h, s, b):
    mu = jnp.mean(h, axis=-1, keepdims=True)
    d = h - mu
    var = jnp.mean(d * d, axis=-1, keepdims=True)
    return jnp.maximum(d * lax.rsqrt(var + 1e-5) * s + b, 0.0)


def _tc_encode(x, W_enc, b_enc, s0, b0):
    def body(x_ref, w_ref, be_ref, s_ref, b_ref, h_ref, t_ref):
        h = jnp.dot(x_ref[...], w_ref[...],
                    preferred_element_type=jnp.float32) + be_ref[...]
        h_ref[...] = h
        t_ref[...] = _ln_relu(h, s_ref[...], b_ref[...])

    full = pl.BlockSpec((H, H), lambda i: (0, 0))
    vec = pl.BlockSpec((1, H), lambda i: (0, 0))
    rows = pl.BlockSpec((_RB, H), lambda i: (i, 0))
    return pl.pallas_call(
        body,
        grid=(N // _RB,),
        in_specs=[rows, full, vec, vec, vec],
        out_specs=[rows, rows],
        out_shape=[jax.ShapeDtypeStruct((N, H), jnp.float32),
                   jax.ShapeDtypeStruct((N, H), jnp.float32)],
    )(x, W_enc, b_enc, s0, b0)


def _agg_from_parts(acc_ref, cnt_ref):
    a = acc_ref[0] + acc_ref[1]
    cnt = cnt_ref[0, :, :1] + cnt_ref[1, :, :1]
    inv = 1.0 / jnp.maximum(cnt, 1.0)
    eps = 1e-7 * (cnt > 0.0).astype(jnp.float32)
    return a * inv + eps


def _tc_layer(acc, cnt, h, Wl_i, bl_i, s_next, b_next):
    def body(acc_ref, cnt_ref, h_ref, w_ref, bv_ref, s_ref, b_ref,
             h_out, t_out):
        agg = _agg_from_parts(acc_ref, cnt_ref)
        hn = h_ref[...] + jnp.dot(agg, w_ref[...],
                                  preferred_element_type=jnp.float32) + bv_ref[...]
        h_out[...] = hn
        t_out[...] = _ln_relu(hn, s_ref[...], b_ref[...])

    rows = pl.BlockSpec((_RB, H), lambda i: (i, 0))
    acc_spec = pl.BlockSpec((NC, _RB, H), lambda i: (0, i, 0))
    cnt_spec = pl.BlockSpec((NC, _RB, H), lambda i: (0, i, 0))
    full = pl.BlockSpec((H, H), lambda i: (0, 0))
    vec = pl.BlockSpec((1, H), lambda i: (0, 0))
    return pl.pallas_call(
        body,
        grid=(N // _RB,),
        in_specs=[acc_spec, cnt_spec, rows, full, vec, vec, vec],
        out_specs=[rows, rows],
        out_shape=[jax.ShapeDtypeStruct((N, H), jnp.float32),
                   jax.ShapeDtypeStruct((N, H), jnp.float32)],
    )(acc, cnt, h, Wl_i, bl_i, s_next, b_next)


def _tc_final_proj(acc, cnt, h, Wl_i, bl_i, sf, bf, W1a, W1b, b1):
    def body(acc_ref, cnt_ref, h_ref, w_ref, bv_ref, s_ref, b_ref,
             w1a_ref, w1b_ref, b1_ref, a_out, b_out):
        agg = _agg_from_parts(acc_ref, cnt_ref)
        hn = h_ref[...] + jnp.dot(agg, w_ref[...],
                                  preferred_element_type=jnp.float32) + bv_ref[...]
        hf = _ln_relu(hn, s_ref[...], b_ref[...])
        a_out[...] = jnp.dot(hf, w1a_ref[...],
                             preferred_element_type=jnp.float32) + b1_ref[...]
        b_out[...] = jnp.dot(hf, w1b_ref[...],
                             preferred_element_type=jnp.float32)

    rows = pl.BlockSpec((_RB, H), lambda i: (i, 0))
    acc_spec = pl.BlockSpec((NC, _RB, H), lambda i: (0, i, 0))
    cnt_spec = pl.BlockSpec((NC, _RB, H), lambda i: (0, i, 0))
    full = pl.BlockSpec((H, H), lambda i: (0, 0))
    vec = pl.BlockSpec((1, H), lambda i: (0, 0))
    return pl.pallas_call(
        body,
        grid=(N // _RB,),
        in_specs=[acc_spec, cnt_spec, rows, full, vec, vec, vec,
                  full, full, vec],
        out_specs=[rows, rows],
        out_shape=[jax.ShapeDtypeStruct((N, H), jnp.float32),
                   jax.ShapeDtypeStruct((N, H), jnp.float32)],
    )(acc, cnt, h, Wl_i, bl_i, sf, bf, W1a, W1b, b1)


def _tc_mlp(r1, r2, W2, b2):
    def body(r1_ref, r2_ref, w_ref, bv_ref, o_ref):
        r = jnp.maximum(r1_ref[...] + r2_ref[...], 0.0)
        o_ref[...] = jnp.dot(r, w_ref[...],
                             preferred_element_type=jnp.float32) + bv_ref[...]

    rows = pl.BlockSpec((_EB, H), lambda i: (i, 0))
    full = pl.BlockSpec((H, H), lambda i: (0, 0))
    vec = pl.BlockSpec((1, H), lambda i: (0, 0))
    return pl.pallas_call(
        body,
        grid=(E // _EB,),
        in_specs=[rows, rows, full, vec],
        out_specs=rows,
        out_shape=jax.ShapeDtypeStruct((E, H), jnp.float32),
    )(r1, r2, W2, b2)


def kernel(x, edge_index, W_enc, b_enc, Wl, bl, ln_s, ln_b, lnf_s, lnf_b,
           W1, b1, W2, b2):
    L = Wl.shape[0]
    src = edge_index[0]
    dst = edge_index[1]
    zeros = jnp.zeros((N, H), jnp.float32)
    ones_tab = jnp.ones((N, H), jnp.float32)
    row = lambda v: v.reshape(1, -1)

    cnt = _sc_segsum(ones_tab, dst, dst, zeros).reshape(NC, N, H)
    h, t = _tc_encode(x, W_enc, row(b_enc), row(ln_s[0]), row(ln_b[0]))
    for i in range(L):
        acc = _sc_segsum(t, src, dst, zeros).reshape(NC, N, H)
        if i + 1 < L:
            h, t = _tc_layer(acc, cnt, h, Wl[i], row(bl[i]),
                             row(ln_s[i + 1]), row(ln_b[i + 1]))
        else:
            a_tab, b_tab = _tc_final_proj(acc, cnt, h, Wl[i], row(bl[i]),
                                          row(lnf_s), row(lnf_b),
                                          W1[:H], W1[H:], row(b1))
    r1, r2 = _sc_gather2(a_tab, b_tab, src, dst)
    return _tc_mlp(r1, r2, W2, row(b2))
